# baseline (speedup 1.0000x reference)
import functools

import jax
import jax.numpy as jnp
from jax import lax
from jax.experimental import pallas as pl
from jax.experimental.pallas import tpu as pltpu
from jax.experimental.pallas import tpu_sc as plsc

BLK = 128
_BF = jnp.bfloat16
_F32 = jnp.float32

_T = 2048
_D = 1024
_FF = 4096
_E = 8
_K = 2
_NW = 32
_TPW = _T // _NW
_NCH = _T // 16
_NB = _T * _K // BLK + (_E - 1)
_P = _NB * BLK
_NBPAD = 48


def _base_up_body(x_ref, wb1_ref, wr_ref, h_ref, logt_ref):
    xb = x_ref[...]
    h = jnp.dot(xb.astype(_BF), wb1_ref[...].astype(_BF),
                preferred_element_type=_F32)
    h_ref[...] = jax.nn.gelu(h).astype(_BF)
    logt_ref[...] = (xb @ wr_ref[...]).T


def _down_body(h_ref, w2_ref, out_ref):
    out_ref[...] = jnp.dot(h_ref[...], w2_ref[...].astype(_BF),
                           preferred_element_type=_F32)


def _moe_up_body(be_ref, xs_ref, we1_ref, h_ref):
    h = jnp.dot(xs_ref[...].astype(_BF), we1_ref[0].astype(_BF),
                preferred_element_type=_F32)
    h_ref[...] = jax.nn.gelu(h).astype(_BF)


def _moe_down_body(be_ref, h_ref, we2_ref, eo_ref):
    eo_ref[...] = jnp.dot(h_ref[...], we2_ref[0].astype(_BF),
                          preferred_element_type=_F32)


def _top2_chunk(lg_v, c):
    off = c * 16
    best = lg_v[0, pl.ds(off, 16)]
    bidx = jnp.zeros((16,), jnp.int32)
    second = jnp.full((16,), -3.0e38, _F32)
    sidx = jnp.zeros((16,), jnp.int32)
    for e in range(1, _E):
        v = lg_v[e, pl.ds(off, 16)]
        ev = jnp.full((16,), e, jnp.int32)
        gt_b = v > best
        gt_s = v > second
        second = jnp.where(gt_b, best, jnp.where(gt_s, v, second))
        sidx = jnp.where(gt_b, bidx, jnp.where(gt_s, ev, sidx))
        best = jnp.where(gt_b, v, best)
        bidx = jnp.where(gt_b, ev, bidx)
    return bidx, sidx, best, second


def _route_body(logt_hbm, x_hbm, xs_hbm, inv_hbm, w_hbm, be_hbm,
                lg_v, xrows_v, idx_v, wv_v, pos_v, be_v, sem):
    wid = lax.axis_index("s") * 2 + lax.axis_index("c")
    tok0 = wid * _TPW
    c0 = wid * (_TPW // 16)

    pltpu.sync_copy(logt_hbm, lg_v)

    one16 = jnp.full((16,), 1, jnp.int32)
    zero16 = jnp.zeros((16,), jnp.int32)

    def i32mask(m):
        return jnp.where(m, one16, zero16)

    def count_body(c, cnt):
        bidx, sidx, _, _ = _top2_chunk(lg_v, c)
        return tuple(
            cnt[e] + i32mask(bidx == e) + i32mask(sidx == e)
            for e in range(_E))

    zeros8 = tuple(jnp.zeros((16,), jnp.int32) for _ in range(_E))
    cnt = lax.fori_loop(0, c0, count_body, zeros8)
    mycnt = [jnp.sum(cnt[e]) for e in range(_E)]

    for j in range(_TPW // 16):
        c = c0 + j
        bidx, sidx, best, second = _top2_chunk(lg_v, c)
        t = jnp.exp(second - best)
        d = 1.0 + t
        idx_v[0, pl.ds(j * 16, 16)] = bidx
        idx_v[1, pl.ds(j * 16, 16)] = sidx
        wv_v[0, pl.ds(j * 16, 16)] = 1.0 / d
        wv_v[1, pl.ds(j * 16, 16)] = t / d
        for s, sv in ((0, bidx), (1, sidx)):
            posv = jnp.zeros((16,), jnp.int32)
            for e in range(_E):
                m = sv == e
                mi = i32mask(m)
                cs = plsc.cumsum(mi)
                posv = jnp.where(
                    m, jnp.full((16,), mycnt[e], jnp.int32) + cs - 1, posv)
                mycnt[e] = mycnt[e] + jnp.sum(mi)
            pos_v[s, pl.ds(j * 16, 16)] = posv
        cnt = tuple(
            cnt[e] + i32mask(bidx == e) + i32mask(sidx == e)
            for e in range(_E))

    cnt = lax.fori_loop(c0 + _TPW // 16, _NCH, count_body, cnt)
    tot = [jnp.sum(cnt[e]) for e in range(_E)]

    offs, nblk = [], []
    run = jnp.int32(0)
    for e in range(_E):
        offs.append(run)
        pb = (tot[e] + (BLK - 1)) // BLK
        nblk.append(pb)
        run = run + pb * BLK

    for j in range(_TPW // 16):
        for s in range(2):
            sv = idx_v[s, pl.ds(j * 16, 16)]
            basev = jnp.zeros((16,), jnp.int32)
            for e in range(_E):
                basev = jnp.where(
                    sv == e, jnp.full((16,), offs[e], jnp.int32), basev)
            pos_v[s, pl.ds(j * 16, 16)] = pos_v[s, pl.ds(j * 16, 16)] + basev

    for s_ in range(_K):
        pltpu.sync_copy(pos_v.at[s_], inv_hbm.at[s_, pl.ds(tok0, _TPW)])
        pltpu.sync_copy(wv_v.at[s_], w_hbm.at[s_, pl.ds(tok0, _TPW)])

    pltpu.sync_copy(x_hbm.at[pl.ds(tok0, _TPW)], xrows_v)
    pltpu.async_copy(xrows_v, xs_hbm.at[pos_v.at[0]], sem).wait()
    pltpu.async_copy(xrows_v, xs_hbm.at[pos_v.at[1]], sem).wait()

    @pl.when(wid == 0)
    def _():
        bend = []
        runb = jnp.int32(0)
        for e in range(_E):
            runb = runb + nblk[e]
            bend.append(runb)
        for g in range(_NBPAD // 16):
            bvec = lax.iota(jnp.int32, 16) + g * 16
            bev = jnp.zeros((16,), jnp.int32)
            for e in range(_E):
                bev = bev + jnp.where(
                    bvec >= jnp.full((16,), bend[e], jnp.int32), one16, zero16)
            be_v[pl.ds(g * 16, 16)] = jnp.minimum(bev, _E - 1)
        pltpu.sync_copy(be_v, be_hbm)


_HG = 32


def _combine_body(base_hbm, eo_hbm, inv_hbm, w_hbm, out_hbm,
                  acc_v, r0_v, r1_v, idx_v, wv_v, sem, sem2):
    wid = lax.axis_index("s") * 2 + lax.axis_index("c")
    tok0 = wid * _TPW
    for s_ in range(_K):
        pltpu.sync_copy(inv_hbm.at[s_, pl.ds(tok0, _TPW)], idx_v.at[s_])
        pltpu.sync_copy(w_hbm.at[s_, pl.ds(tok0, _TPW)], wv_v.at[s_])
    for g in range(_TPW // _HG):
        t0 = tok0 + g * _HG
        pltpu.sync_copy(base_hbm.at[pl.ds(t0, _HG)], acc_v)
        cp0 = pltpu.async_copy(
            eo_hbm.at[idx_v.at[0, pl.ds(g * _HG, _HG)]], r0_v, sem)
        cp1 = pltpu.async_copy(
            eo_hbm.at[idx_v.at[1, pl.ds(g * _HG, _HG)]], r1_v, sem2)
        cp0.wait()
        cp1.wait()
        for q in range(_HG // 16):
            wc0 = wv_v[0, pl.ds(g * _HG + q * 16, 16)]
            wc1 = wv_v[1, pl.ds(g * _HG + q * 16, 16)]
            for l in range(16):
                j = q * 16 + l
                w0 = wc0[l]
                w1 = wc1[l]

                def cbody(cc, carry, j=j, w0=w0, w1=w1):
                    for u in range(8):
                        sl = pl.ds(cc * 128 + u * 16, 16)
                        acc_v[j, sl] = (acc_v[j, sl] + w0 * r0_v[j, sl]
                                        + w1 * r1_v[j, sl])
                    return carry
                lax.fori_loop(0, _D // 128, cbody, 0)
        pltpu.sync_copy(acc_v, out_hbm.at[pl.ds(t0, _HG)])


_combine = functools.partial(
    pl.kernel,
    out_type=[jax.ShapeDtypeStruct((_T, _D), _F32)],
    mesh=plsc.VectorSubcoreMesh(core_axis_name="c", subcore_axis_name="s"),
    scratch_types=[
        pltpu.VMEM((_HG, _D), _F32),
        pltpu.VMEM((_HG, _D), _F32),
        pltpu.VMEM((_HG, _D), _F32),
        pltpu.VMEM((_K, _TPW), jnp.int32),
        pltpu.VMEM((_K, _TPW), _F32),
        pltpu.SemaphoreType.DMA,
        pltpu.SemaphoreType.DMA,
    ],
    compiler_params=pltpu.CompilerParams(needs_layout_passes=False),
)(_combine_body)


_route = functools.partial(
    pl.kernel,
    out_type=[
        jax.ShapeDtypeStruct((_P, _D), _F32),
        jax.ShapeDtypeStruct((_K, _T), jnp.int32),
        jax.ShapeDtypeStruct((_K, _T), _F32),
        jax.ShapeDtypeStruct((_NBPAD,), jnp.int32),
    ],
    mesh=plsc.VectorSubcoreMesh(core_axis_name="c", subcore_axis_name="s"),
    scratch_types=[
        pltpu.VMEM((_E, _T), _F32),
        pltpu.VMEM((_TPW, _D), _F32),
        pltpu.VMEM((_K, _TPW), jnp.int32),
        pltpu.VMEM((_K, _TPW), _F32),
        pltpu.VMEM((_K, _TPW), jnp.int32),
        pltpu.VMEM((_NBPAD,), jnp.int32),
        pltpu.SemaphoreType.DMA,
    ],
    compiler_params=pltpu.CompilerParams(needs_layout_passes=False),
)(_route_body)


def kernel(hidden_states, Wb1, Wb2, Wr, We1, We2):
    B, S, D = hidden_states.shape
    E, _, FF = We1.shape
    T = B * S
    x = hidden_states.reshape(T, D)

    n_tb = T // BLK
    h_b, logt = pl.pallas_call(
        _base_up_body,
        grid=(n_tb,),
        in_specs=[
            pl.BlockSpec((BLK, D), lambda i: (i, 0)),
            pl.BlockSpec((D, FF), lambda i: (0, 0)),
            pl.BlockSpec((D, E), lambda i: (0, 0)),
        ],
        out_specs=[
            pl.BlockSpec((BLK, FF), lambda i: (i, 0)),
            pl.BlockSpec((E, BLK), lambda i: (0, i)),
        ],
        out_shape=[
            jax.ShapeDtypeStruct((T, FF), _BF),
            jax.ShapeDtypeStruct((E, T), jnp.float32),
        ],
    )(x, Wb1, Wr)

    xs, invT, wT, block_expert = _route(logt, x)

    base = pl.pallas_call(
        _down_body,
        grid=(n_tb,),
        in_specs=[
            pl.BlockSpec((BLK, FF), lambda i: (i, 0)),
            pl.BlockSpec((FF, D), lambda i: (0, 0)),
        ],
        out_specs=pl.BlockSpec((BLK, D), lambda i: (i, 0)),
        out_shape=jax.ShapeDtypeStruct((T, D), jnp.float32),
    )(h_b, Wb2)

    up_spec = pltpu.PrefetchScalarGridSpec(
        num_scalar_prefetch=1,
        grid=(_NB,),
        in_specs=[
            pl.BlockSpec((BLK, D), lambda b, be: (b, 0)),
            pl.BlockSpec((1, D, FF), lambda b, be: (be[b], 0, 0)),
        ],
        out_specs=pl.BlockSpec((BLK, FF), lambda b, be: (b, 0)),
    )
    h = pl.pallas_call(
        _moe_up_body,
        grid_spec=up_spec,
        out_shape=jax.ShapeDtypeStruct((_P, FF), _BF),
    )(block_expert, xs, We1)

    down_spec = pltpu.PrefetchScalarGridSpec(
        num_scalar_prefetch=1,
        grid=(_NB,),
        in_specs=[
            pl.BlockSpec((BLK, FF), lambda b, be: (b, 0)),
            pl.BlockSpec((1, FF, D), lambda b, be: (be[b], 0, 0)),
        ],
        out_specs=pl.BlockSpec((BLK, D), lambda b, be: (b, 0)),
    )
    eo = pl.pallas_call(
        _moe_down_body,
        grid_spec=down_spec,
        out_shape=jax.ShapeDtypeStruct((_P, D), jnp.float32),
    )(block_expert, h, We2)

    (out,) = _combine(base, eo, invT, wT)
    return out.reshape(B, S, D)


# speedup vs baseline: 1.0326x; 1.0326x over previous
import functools

import jax
import jax.numpy as jnp
from jax import lax
from jax.experimental import pallas as pl
from jax.experimental.pallas import tpu as pltpu
from jax.experimental.pallas import tpu_sc as plsc

BLK = 256
_BF = jnp.bfloat16
_F32 = jnp.float32

_T = 2048
_D = 1024
_FF = 4096
_E = 8
_K = 2
_NW = 32
_TPW = _T // _NW
_NCH = _T // 16
_NB = _T * _K // BLK + (_E - 1)
_P = _NB * BLK
_NBPAD = 32


def _base_body(x_ref, wb1_ref, wb2_ref, wr_ref, out_ref, logt_ref):
    xb = x_ref[...]
    h = jnp.dot(xb.astype(_BF), wb1_ref[...].astype(_BF),
                preferred_element_type=_F32)
    hb = jax.nn.gelu(h).astype(_BF)
    out_ref[...] = jnp.dot(hb, wb2_ref[...].astype(_BF),
                           preferred_element_type=_F32)
    logt_ref[...] = (xb @ wr_ref[...]).T


def _moe_up_body(be_ref, xs_ref, we1_ref, h_ref):
    h = jnp.dot(xs_ref[...].astype(_BF), we1_ref[0].astype(_BF),
                preferred_element_type=_F32)
    h_ref[...] = jax.nn.gelu(h).astype(_BF)


def _moe_down_body(be_ref, h_ref, we2_ref, eo_ref):
    eo_ref[...] = jnp.dot(h_ref[...], we2_ref[0].astype(_BF),
                          preferred_element_type=_F32)


def _top2_chunk(lg_v, c):
    off = c * 16
    best = lg_v[0, pl.ds(off, 16)]
    bidx = jnp.zeros((16,), jnp.int32)
    second = jnp.full((16,), -3.0e38, _F32)
    sidx = jnp.zeros((16,), jnp.int32)
    for e in range(1, _E):
        v = lg_v[e, pl.ds(off, 16)]
        ev = jnp.full((16,), e, jnp.int32)
        gt_b = v > best
        gt_s = v > second
        second = jnp.where(gt_b, best, jnp.where(gt_s, v, second))
        sidx = jnp.where(gt_b, bidx, jnp.where(gt_s, ev, sidx))
        best = jnp.where(gt_b, v, best)
        bidx = jnp.where(gt_b, ev, bidx)
    return bidx, sidx, best, second


def _route_body(logt_hbm, x_hbm, xs_hbm, inv_hbm, w_hbm, be_hbm,
                lg_v, xrows_v, idx_v, wv_v, pos_v, be_v, sem):
    wid = lax.axis_index("s") * 2 + lax.axis_index("c")
    tok0 = wid * _TPW
    c0 = wid * (_TPW // 16)

    pltpu.sync_copy(logt_hbm, lg_v)

    one16 = jnp.full((16,), 1, jnp.int32)
    zero16 = jnp.zeros((16,), jnp.int32)

    def i32mask(m):
        return jnp.where(m, one16, zero16)

    def count_body(c, cnt):
        bidx, sidx, _, _ = _top2_chunk(lg_v, c)
        return tuple(
            cnt[e] + i32mask(bidx == e) + i32mask(sidx == e)
            for e in range(_E))

    zeros8 = tuple(jnp.zeros((16,), jnp.int32) for _ in range(_E))
    cnt = lax.fori_loop(0, c0, count_body, zeros8)
    mycnt = [jnp.sum(cnt[e]) for e in range(_E)]

    for j in range(_TPW // 16):
        c = c0 + j
        bidx, sidx, best, second = _top2_chunk(lg_v, c)
        t = jnp.exp(second - best)
        d = 1.0 + t
        idx_v[0, pl.ds(j * 16, 16)] = bidx
        idx_v[1, pl.ds(j * 16, 16)] = sidx
        wv_v[0, pl.ds(j * 16, 16)] = 1.0 / d
        wv_v[1, pl.ds(j * 16, 16)] = t / d
        for s, sv in ((0, bidx), (1, sidx)):
            posv = jnp.zeros((16,), jnp.int32)
            for e in range(_E):
                m = sv == e
                mi = i32mask(m)
                cs = plsc.cumsum(mi)
                posv = jnp.where(
                    m, jnp.full((16,), mycnt[e], jnp.int32) + cs - 1, posv)
                mycnt[e] = mycnt[e] + jnp.sum(mi)
            pos_v[s, pl.ds(j * 16, 16)] = posv
        cnt = tuple(
            cnt[e] + i32mask(bidx == e) + i32mask(sidx == e)
            for e in range(_E))

    cnt = lax.fori_loop(c0 + _TPW // 16, _NCH, count_body, cnt)
    tot = [jnp.sum(cnt[e]) for e in range(_E)]

    offs, nblk = [], []
    run = jnp.int32(0)
    for e in range(_E):
        offs.append(run)
        pb = (tot[e] + (BLK - 1)) // BLK
        nblk.append(pb)
        run = run + pb * BLK

    for j in range(_TPW // 16):
        for s in range(2):
            sv = idx_v[s, pl.ds(j * 16, 16)]
            basev = jnp.zeros((16,), jnp.int32)
            for e in range(_E):
                basev = jnp.where(
                    sv == e, jnp.full((16,), offs[e], jnp.int32), basev)
            pos_v[s, pl.ds(j * 16, 16)] = pos_v[s, pl.ds(j * 16, 16)] + basev

    for s_ in range(_K):
        pltpu.sync_copy(pos_v.at[s_], inv_hbm.at[s_, pl.ds(tok0, _TPW)])
        pltpu.sync_copy(wv_v.at[s_], w_hbm.at[s_, pl.ds(tok0, _TPW)])

    pltpu.sync_copy(x_hbm.at[pl.ds(tok0, _TPW)], xrows_v)
    pltpu.async_copy(xrows_v, xs_hbm.at[pos_v.at[0]], sem).wait()
    pltpu.async_copy(xrows_v, xs_hbm.at[pos_v.at[1]], sem).wait()

    @pl.when(wid == 0)
    def _():
        bend = []
        runb = jnp.int32(0)
        for e in range(_E):
            runb = runb + nblk[e]
            bend.append(runb)
        for g in range(_NBPAD // 16):
            bvec = lax.iota(jnp.int32, 16) + g * 16
            bev = jnp.zeros((16,), jnp.int32)
            for e in range(_E):
                bev = bev + jnp.where(
                    bvec >= jnp.full((16,), bend[e], jnp.int32), one16, zero16)
            be_v[pl.ds(g * 16, 16)] = jnp.minimum(bev, _E - 1)
        pltpu.sync_copy(be_v, be_hbm)


_HG = 32


def _combine_body(base_hbm, eo_hbm, inv_hbm, w_hbm, out_hbm,
                  acc_v, r0_v, r1_v, idx_v, wv_v, sem, sem2):
    wid = lax.axis_index("s") * 2 + lax.axis_index("c")
    tok0 = wid * _TPW
    for s_ in range(_K):
        pltpu.sync_copy(inv_hbm.at[s_, pl.ds(tok0, _TPW)], idx_v.at[s_])
        pltpu.sync_copy(w_hbm.at[s_, pl.ds(tok0, _TPW)], wv_v.at[s_])
    for g in range(_TPW // _HG):
        t0 = tok0 + g * _HG
        pltpu.sync_copy(base_hbm.at[pl.ds(t0, _HG)], acc_v)
        cp0 = pltpu.async_copy(
            eo_hbm.at[idx_v.at[0, pl.ds(g * _HG, _HG)]], r0_v, sem)
        cp1 = pltpu.async_copy(
            eo_hbm.at[idx_v.at[1, pl.ds(g * _HG, _HG)]], r1_v, sem2)
        cp0.wait()
        cp1.wait()
        for q in range(_HG // 16):
            wc0 = wv_v[0, pl.ds(g * _HG + q * 16, 16)]
            wc1 = wv_v[1, pl.ds(g * _HG + q * 16, 16)]
            for l in range(16):
                j = q * 16 + l
                w0 = wc0[l]
                w1 = wc1[l]

                def cbody(cc, carry, j=j, w0=w0, w1=w1):
                    for u in range(8):
                        sl = pl.ds(cc * 128 + u * 16, 16)
                        acc_v[j, sl] = (acc_v[j, sl] + w0 * r0_v[j, sl]
                                        + w1 * r1_v[j, sl])
                    return carry
                lax.fori_loop(0, _D // 128, cbody, 0)
        pltpu.sync_copy(acc_v, out_hbm.at[pl.ds(t0, _HG)])


_combine = functools.partial(
    pl.kernel,
    out_type=[jax.ShapeDtypeStruct((_T, _D), _F32)],
    mesh=plsc.VectorSubcoreMesh(core_axis_name="c", subcore_axis_name="s"),
    scratch_types=[
        pltpu.VMEM((_HG, _D), _F32),
        pltpu.VMEM((_HG, _D), _F32),
        pltpu.VMEM((_HG, _D), _F32),
        pltpu.VMEM((_K, _TPW), jnp.int32),
        pltpu.VMEM((_K, _TPW), _F32),
        pltpu.SemaphoreType.DMA,
        pltpu.SemaphoreType.DMA,
    ],
    compiler_params=pltpu.CompilerParams(needs_layout_passes=False),
)(_combine_body)


_route = functools.partial(
    pl.kernel,
    out_type=[
        jax.ShapeDtypeStruct((_P, _D), _F32),
        jax.ShapeDtypeStruct((_K, _T), jnp.int32),
        jax.ShapeDtypeStruct((_K, _T), _F32),
        jax.ShapeDtypeStruct((_NBPAD,), jnp.int32),
    ],
    mesh=plsc.VectorSubcoreMesh(core_axis_name="c", subcore_axis_name="s"),
    scratch_types=[
        pltpu.VMEM((_E, _T), _F32),
        pltpu.VMEM((_TPW, _D), _F32),
        pltpu.VMEM((_K, _TPW), jnp.int32),
        pltpu.VMEM((_K, _TPW), _F32),
        pltpu.VMEM((_K, _TPW), jnp.int32),
        pltpu.VMEM((_NBPAD,), jnp.int32),
        pltpu.SemaphoreType.DMA,
    ],
    compiler_params=pltpu.CompilerParams(needs_layout_passes=False),
)(_route_body)


def kernel(hidden_states, Wb1, Wb2, Wr, We1, We2):
    B, S, D = hidden_states.shape
    E, _, FF = We1.shape
    T = B * S
    x = hidden_states.reshape(T, D)

    n_tb = T // BLK
    base, logt = pl.pallas_call(
        _base_body,
        grid=(n_tb,),
        in_specs=[
            pl.BlockSpec((BLK, D), lambda i: (i, 0)),
            pl.BlockSpec((D, FF), lambda i: (0, 0)),
            pl.BlockSpec((FF, D), lambda i: (0, 0)),
            pl.BlockSpec((D, E), lambda i: (0, 0)),
        ],
        out_specs=[
            pl.BlockSpec((BLK, D), lambda i: (i, 0)),
            pl.BlockSpec((E, BLK), lambda i: (0, i)),
        ],
        out_shape=[
            jax.ShapeDtypeStruct((T, D), jnp.float32),
            jax.ShapeDtypeStruct((E, T), jnp.float32),
        ],
        compiler_params=pltpu.CompilerParams(
            vmem_limit_bytes=64 * 1024 * 1024),
    )(x, Wb1, Wb2, Wr)

    xs, invT, wT, block_expert = _route(logt, x)

    up_spec = pltpu.PrefetchScalarGridSpec(
        num_scalar_prefetch=1,
        grid=(_NB,),
        in_specs=[
            pl.BlockSpec((BLK, D), lambda b, be: (b, 0)),
            pl.BlockSpec((1, D, FF), lambda b, be: (be[b], 0, 0)),
        ],
        out_specs=pl.BlockSpec((BLK, FF), lambda b, be: (b, 0)),
    )
    h = pl.pallas_call(
        _moe_up_body,
        grid_spec=up_spec,
        out_shape=jax.ShapeDtypeStruct((_P, FF), _BF),
    )(block_expert, xs, We1)

    down_spec = pltpu.PrefetchScalarGridSpec(
        num_scalar_prefetch=1,
        grid=(_NB,),
        in_specs=[
            pl.BlockSpec((BLK, FF), lambda b, be: (b, 0)),
            pl.BlockSpec((1, FF, D), lambda b, be: (be[b], 0, 0)),
        ],
        out_specs=pl.BlockSpec((BLK, D), lambda b, be: (b, 0)),
    )
    eo = pl.pallas_call(
        _moe_down_body,
        grid_spec=down_spec,
        out_shape=jax.ShapeDtypeStruct((_P, D), jnp.float32),
    )(block_expert, h, We2)

    (out,) = _combine(base, eo, invT, wT)
    return out.reshape(B, S, D)


# speedup vs baseline: 1.0557x; 1.0224x over previous
import functools

import jax
import jax.numpy as jnp
from jax import lax
from jax.experimental import pallas as pl
from jax.experimental.pallas import tpu as pltpu
from jax.experimental.pallas import tpu_sc as plsc

BLK = 256
_BF = jnp.bfloat16
_F32 = jnp.float32

_T = 2048
_D = 1024
_FF = 4096
_E = 8
_K = 2
_NW = 32
_TPW = _T // _NW
_NCH = _T // 16
_NB = _T * _K // BLK + (_E - 1)
_P = _NB * BLK
_NBPAD = 32


def _logit_body(x_ref, wr_ref, logt_ref):
    logt_ref[...] = (x_ref[...] @ wr_ref[...]).T


def _base_body(x_ref, wb1_ref, wb2_ref, out_ref):
    xb = x_ref[...]
    h = jnp.dot(xb.astype(_BF), wb1_ref[...].astype(_BF),
                preferred_element_type=_F32)
    hb = jax.nn.gelu(h).astype(_BF)
    out_ref[...] = jnp.dot(hb, wb2_ref[...].astype(_BF),
                           preferred_element_type=_F32)


def _moe_up_body(be_ref, xs_ref, we1_ref, h_ref):
    h = jnp.dot(xs_ref[...].astype(_BF), we1_ref[0].astype(_BF),
                preferred_element_type=_F32)
    h_ref[...] = jax.nn.gelu(h).astype(_BF)


def _moe_down_body(be_ref, h_ref, we2_ref, eo_ref):
    eo_ref[...] = jnp.dot(h_ref[...], we2_ref[0].astype(_BF),
                          preferred_element_type=_F32)


def _top2_chunk(lg_v, c):
    off = c * 16
    best = lg_v[0, pl.ds(off, 16)]
    bidx = jnp.zeros((16,), jnp.int32)
    second = jnp.full((16,), -3.0e38, _F32)
    sidx = jnp.zeros((16,), jnp.int32)
    for e in range(1, _E):
        v = lg_v[e, pl.ds(off, 16)]
        ev = jnp.full((16,), e, jnp.int32)
        gt_b = v > best
        gt_s = v > second
        second = jnp.where(gt_b, best, jnp.where(gt_s, v, second))
        sidx = jnp.where(gt_b, bidx, jnp.where(gt_s, ev, sidx))
        best = jnp.where(gt_b, v, best)
        bidx = jnp.where(gt_b, ev, bidx)
    return bidx, sidx, best, second


def _route_body(logt_hbm, x_hbm, xs_hbm, inv_hbm, w_hbm, be_hbm,
                lg_v, xrows_v, idx_v, wv_v, pos_v, be_v, sem):
    wid = lax.axis_index("s") * 2 + lax.axis_index("c")
    tok0 = wid * _TPW
    c0 = wid * (_TPW // 16)

    pltpu.sync_copy(logt_hbm, lg_v)

    one16 = jnp.full((16,), 1, jnp.int32)
    zero16 = jnp.zeros((16,), jnp.int32)

    def i32mask(m):
        return jnp.where(m, one16, zero16)

    def count_body(c, cnt):
        bidx, sidx, _, _ = _top2_chunk(lg_v, c)
        return tuple(
            cnt[e] + i32mask(bidx == e) + i32mask(sidx == e)
            for e in range(_E))

    zeros8 = tuple(jnp.zeros((16,), jnp.int32) for _ in range(_E))
    cnt = lax.fori_loop(0, c0, count_body, zeros8)
    mycnt = [jnp.sum(cnt[e]) for e in range(_E)]

    for j in range(_TPW // 16):
        c = c0 + j
        bidx, sidx, best, second = _top2_chunk(lg_v, c)
        t = jnp.exp(second - best)
        d = 1.0 + t
        idx_v[0, pl.ds(j * 16, 16)] = bidx
        idx_v[1, pl.ds(j * 16, 16)] = sidx
        wv_v[0, pl.ds(j * 16, 16)] = 1.0 / d
        wv_v[1, pl.ds(j * 16, 16)] = t / d
        for s, sv in ((0, bidx), (1, sidx)):
            posv = jnp.zeros((16,), jnp.int32)
            for e in range(_E):
                m = sv == e
                mi = i32mask(m)
                cs = plsc.cumsum(mi)
                posv = jnp.where(
                    m, jnp.full((16,), mycnt[e], jnp.int32) + cs - 1, posv)
                mycnt[e] = mycnt[e] + jnp.sum(mi)
            pos_v[s, pl.ds(j * 16, 16)] = posv
        cnt = tuple(
            cnt[e] + i32mask(bidx == e) + i32mask(sidx == e)
            for e in range(_E))

    cnt = lax.fori_loop(c0 + _TPW // 16, _NCH, count_body, cnt)
    tot = [jnp.sum(cnt[e]) for e in range(_E)]

    offs, nblk = [], []
    run = jnp.int32(0)
    for e in range(_E):
        offs.append(run)
        pb = (tot[e] + (BLK - 1)) // BLK
        nblk.append(pb)
        run = run + pb * BLK

    for j in range(_TPW // 16):
        for s in range(2):
            sv = idx_v[s, pl.ds(j * 16, 16)]
            basev = jnp.zeros((16,), jnp.int32)
            for e in range(_E):
                basev = jnp.where(
                    sv == e, jnp.full((16,), offs[e], jnp.int32), basev)
            pos_v[s, pl.ds(j * 16, 16)] = pos_v[s, pl.ds(j * 16, 16)] + basev

    for s_ in range(_K):
        pltpu.sync_copy(pos_v.at[s_], inv_hbm.at[s_, pl.ds(tok0, _TPW)])
        pltpu.sync_copy(wv_v.at[s_], w_hbm.at[s_, pl.ds(tok0, _TPW)])

    pltpu.sync_copy(x_hbm.at[pl.ds(tok0, _TPW)], xrows_v)
    pltpu.async_copy(xrows_v, xs_hbm.at[pos_v.at[0]], sem).wait()
    pltpu.async_copy(xrows_v, xs_hbm.at[pos_v.at[1]], sem).wait()

    @pl.when(wid == 0)
    def _():
        bend = []
        runb = jnp.int32(0)
        for e in range(_E):
            runb = runb + nblk[e]
            bend.append(runb)
        for g in range(_NBPAD // 16):
            bvec = lax.iota(jnp.int32, 16) + g * 16
            bev = jnp.zeros((16,), jnp.int32)
            for e in range(_E):
                bev = bev + jnp.where(
                    bvec >= jnp.full((16,), bend[e], jnp.int32), one16, zero16)
            be_v[pl.ds(g * 16, 16)] = jnp.minimum(bev, _E - 1)
        pltpu.sync_copy(be_v, be_hbm)


_HG = 32


def _combine_body(base_hbm, eo_hbm, inv_hbm, w_hbm, out_hbm,
                  acc_v, r0_v, r1_v, idx_v, wv_v, sem, sem2):
    wid = lax.axis_index("s") * 2 + lax.axis_index("c")
    tok0 = wid * _TPW
    for s_ in range(_K):
        pltpu.sync_copy(inv_hbm.at[s_, pl.ds(tok0, _TPW)], idx_v.at[s_])
        pltpu.sync_copy(w_hbm.at[s_, pl.ds(tok0, _TPW)], wv_v.at[s_])
    for g in range(_TPW // _HG):
        t0 = tok0 + g * _HG
        pltpu.sync_copy(base_hbm.at[pl.ds(t0, _HG)], acc_v)
        cp0 = pltpu.async_copy(
            eo_hbm.at[idx_v.at[0, pl.ds(g * _HG, _HG)]], r0_v, sem)
        cp1 = pltpu.async_copy(
            eo_hbm.at[idx_v.at[1, pl.ds(g * _HG, _HG)]], r1_v, sem2)
        cp0.wait()
        cp1.wait()
        for q in range(_HG // 16):
            wc0 = wv_v[0, pl.ds(g * _HG + q * 16, 16)]
            wc1 = wv_v[1, pl.ds(g * _HG + q * 16, 16)]
            for l in range(16):
                j = q * 16 + l
                w0 = wc0[l]
                w1 = wc1[l]

                def cbody(cc, carry, j=j, w0=w0, w1=w1):
                    for u in range(8):
                        sl = pl.ds(cc * 128 + u * 16, 16)
                        acc_v[j, sl] = (acc_v[j, sl] + w0 * r0_v[j, sl]
                                        + w1 * r1_v[j, sl])
                    return carry
                lax.fori_loop(0, _D // 128, cbody, 0)
        pltpu.sync_copy(acc_v, out_hbm.at[pl.ds(t0, _HG)])


_combine = functools.partial(
    pl.kernel,
    out_type=[jax.ShapeDtypeStruct((_T, _D), _F32)],
    mesh=plsc.VectorSubcoreMesh(core_axis_name="c", subcore_axis_name="s"),
    scratch_types=[
        pltpu.VMEM((_HG, _D), _F32),
        pltpu.VMEM((_HG, _D), _F32),
        pltpu.VMEM((_HG, _D), _F32),
        pltpu.VMEM((_K, _TPW), jnp.int32),
        pltpu.VMEM((_K, _TPW), _F32),
        pltpu.SemaphoreType.DMA,
        pltpu.SemaphoreType.DMA,
    ],
    compiler_params=pltpu.CompilerParams(needs_layout_passes=False),
)(_combine_body)


_route = functools.partial(
    pl.kernel,
    out_type=[
        jax.ShapeDtypeStruct((_P, _D), _F32),
        jax.ShapeDtypeStruct((_K, _T), jnp.int32),
        jax.ShapeDtypeStruct((_K, _T), _F32),
        jax.ShapeDtypeStruct((_NBPAD,), jnp.int32),
    ],
    mesh=plsc.VectorSubcoreMesh(core_axis_name="c", subcore_axis_name="s"),
    scratch_types=[
        pltpu.VMEM((_E, _T), _F32),
        pltpu.VMEM((_TPW, _D), _F32),
        pltpu.VMEM((_K, _TPW), jnp.int32),
        pltpu.VMEM((_K, _TPW), _F32),
        pltpu.VMEM((_K, _TPW), jnp.int32),
        pltpu.VMEM((_NBPAD,), jnp.int32),
        pltpu.SemaphoreType.DMA,
    ],
    compiler_params=pltpu.CompilerParams(needs_layout_passes=False),
)(_route_body)


def kernel(hidden_states, Wb1, Wb2, Wr, We1, We2):
    B, S, D = hidden_states.shape
    E, _, FF = We1.shape
    T = B * S
    x = hidden_states.reshape(T, D)

    n_tb = T // BLK
    logt = pl.pallas_call(
        _logit_body,
        grid=(1,),
        in_specs=[
            pl.BlockSpec((T, D), lambda i: (0, 0)),
            pl.BlockSpec((D, E), lambda i: (0, 0)),
        ],
        out_specs=pl.BlockSpec((E, T), lambda i: (0, 0)),
        out_shape=jax.ShapeDtypeStruct((E, T), jnp.float32),
    )(x, Wr)

    xs, invT, wT, block_expert = _route(logt, x)

    base = pl.pallas_call(
        _base_body,
        grid=(n_tb,),
        in_specs=[
            pl.BlockSpec((BLK, D), lambda i: (i, 0)),
            pl.BlockSpec((D, FF), lambda i: (0, 0)),
            pl.BlockSpec((FF, D), lambda i: (0, 0)),
        ],
        out_specs=pl.BlockSpec((BLK, D), lambda i: (i, 0)),
        out_shape=jax.ShapeDtypeStruct((T, D), jnp.float32),
        compiler_params=pltpu.CompilerParams(
            vmem_limit_bytes=64 * 1024 * 1024),
    )(x, Wb1, Wb2)

    up_spec = pltpu.PrefetchScalarGridSpec(
        num_scalar_prefetch=1,
        grid=(_NB,),
        in_specs=[
            pl.BlockSpec((BLK, D), lambda b, be: (b, 0)),
            pl.BlockSpec((1, D, FF), lambda b, be: (be[b], 0, 0)),
        ],
        out_specs=pl.BlockSpec((BLK, FF), lambda b, be: (b, 0)),
    )
    h = pl.pallas_call(
        _moe_up_body,
        grid_spec=up_spec,
        out_shape=jax.ShapeDtypeStruct((_P, FF), _BF),
    )(block_expert, xs, We1)

    down_spec = pltpu.PrefetchScalarGridSpec(
        num_scalar_prefetch=1,
        grid=(_NB,),
        in_specs=[
            pl.BlockSpec((BLK, FF), lambda b, be: (b, 0)),
            pl.BlockSpec((1, FF, D), lambda b, be: (be[b], 0, 0)),
        ],
        out_specs=pl.BlockSpec((BLK, D), lambda b, be: (b, 0)),
    )
    eo = pl.pallas_call(
        _moe_down_body,
        grid_spec=down_spec,
        out_shape=jax.ShapeDtypeStruct((_P, D), jnp.float32),
    )(block_expert, h, We2)

    (out,) = _combine(base, eo, invT, wT)
    return out.reshape(B, S, D)
